# baseline (speedup 1.0000x reference)
import functools

import jax
import jax.numpy as jnp
from jax import lax
from jax.experimental import pallas as pl
from jax.experimental.pallas import tpu as pltpu
from jax.experimental.pallas import tpu_sc as plsc

_NC = 2
_NS = 16
_NW = _NC * _NS

_PACK = 4


@functools.partial(jax.jit, static_argnums=(4, 5, 6))
def _gather_sc(ug, ig, ut4, it4, V4, B, D4):
    b_per_w = B // _NW
    chunk = 256
    n_chunks = b_per_w // chunk
    mesh = plsc.VectorSubcoreMesh(core_axis_name="c", subcore_axis_name="s")

    @functools.partial(
        pl.kernel,
        mesh=mesh,
        compiler_params=pltpu.CompilerParams(use_tc_tiling_on_sc=True),
        out_type=[
            jax.ShapeDtypeStruct((B, D4), jnp.float32),
            jax.ShapeDtypeStruct((B, D4), jnp.float32),
        ],
        scratch_types=[
            pltpu.VMEM((b_per_w,), jnp.int32),
            pltpu.VMEM((b_per_w,), jnp.int32),
            pltpu.VMEM((chunk, D4), jnp.float32),
            pltpu.VMEM((chunk, D4), jnp.float32),
            pltpu.SemaphoreType.DMA,
            pltpu.SemaphoreType.DMA,
        ],
    )
    def k(ug_hbm, ig_hbm, ut_hbm, it_hbm, ue_hbm, ie_hbm,
          ugx, igx, ubuf, ibuf, sem_u, sem_i):
        wid = lax.axis_index("s") * _NC + lax.axis_index("c")
        base = wid * b_per_w

        pltpu.sync_copy(ug_hbm.at[pl.ds(base, b_per_w)], ugx)
        pltpu.sync_copy(ig_hbm.at[pl.ds(base, b_per_w)], igx)

        for h in range(n_chunks):
            cp_u = pltpu.async_copy(
                ut_hbm.at[ugx.at[pl.ds(h * chunk, chunk)]], ubuf, sem_u)
            cp_i = pltpu.async_copy(
                it_hbm.at[igx.at[pl.ds(h * chunk, chunk)]], ibuf, sem_i)
            cp_u.wait()
            cp_i.wait()
            pltpu.sync_copy(ubuf, ue_hbm.at[pl.ds(base + h * chunk, chunk)])
            pltpu.sync_copy(ibuf, ie_hbm.at[pl.ds(base + h * chunk, chunk)])

    return k(ug, ig, ut4, it4)


def _select_dot_body(uq_ref, iq_ref, u4_ref, i4_ref, b_ref,
                     o_ref, ue_ref, ie_ref):
    d = ue_ref.shape[1]
    u4 = u4_ref[...]
    i4 = i4_ref[...]
    uq = uq_ref[...]
    iq = iq_ref[...]
    ue = jnp.zeros_like(ue_ref)
    ie = jnp.zeros_like(ie_ref)
    for q in range(_PACK):
        ue = ue + jnp.where(uq == q, u4[:, q * d:(q + 1) * d], 0.0)
        ie = ie + jnp.where(iq == q, i4[:, q * d:(q + 1) * d], 0.0)
    ue_ref[...] = ue
    ie_ref[...] = ie
    o_ref[...] = jnp.sum(ue * ie, axis=1, keepdims=True) + b_ref[0]


@functools.partial(jax.jit, static_argnums=(5, 6, 7, 8))
def _select_dot_tc(uq, iq, ue4, ie4, bias, B, D, D4, blk):
    return pl.pallas_call(
        _select_dot_body,
        grid=(B // blk,),
        in_specs=[
            pl.BlockSpec((blk, 1), lambda i: (i, 0)),
            pl.BlockSpec((blk, 1), lambda i: (i, 0)),
            pl.BlockSpec((blk, D4), lambda i: (i, 0)),
            pl.BlockSpec((blk, D4), lambda i: (i, 0)),
            pl.BlockSpec(memory_space=pltpu.SMEM),
        ],
        out_specs=[
            pl.BlockSpec((blk, 1), lambda i: (i, 0)),
            pl.BlockSpec((blk, D), lambda i: (i, 0)),
            pl.BlockSpec((blk, D), lambda i: (i, 0)),
        ],
        out_shape=[
            jax.ShapeDtypeStruct((B, 1), jnp.float32),
            jax.ShapeDtypeStruct((B, D), jnp.float32),
            jax.ShapeDtypeStruct((B, D), jnp.float32),
        ],
    )(uq, iq, ue4, ie4, bias)


def kernel(x, user_table, item_table, bias):
    B = x.shape[0]
    V, D = user_table.shape
    D4 = _PACK * D
    uidx = x[:, 0]
    iidx = x[:, 1]
    ue4, ie4 = _gather_sc(
        uidx // _PACK, iidx // _PACK,
        user_table.reshape(V // _PACK, D4), item_table.reshape(V // _PACK, D4),
        V // _PACK, B, D4)
    out, ue, ie = _select_dot_tc(
        (uidx % _PACK)[:, None], (iidx % _PACK)[:, None], ue4, ie4, bias,
        B, D, D4, 2048)
    return out, ue, ie


# speedup vs baseline: 2.5701x; 2.5701x over previous
import functools

import jax
import jax.numpy as jnp
from jax import lax
from jax.experimental import pallas as pl
from jax.experimental.pallas import tpu as pltpu
from jax.experimental.pallas import tpu_sc as plsc

_NC = 2
_NS = 16
_NW = _NC * _NS
_SUB = 8
_CHUNK = 256


def _extract(vec, i):
    return jnp.squeeze(lax.slice(vec, (i,), (i + 1,)))


@functools.partial(jax.jit, static_argnums=(4, 5, 6))
def _gather_sc(uidx, iidx, ut3, it3, V, B, D):
    b_per_w = B // _NW
    n_chunks = b_per_w // _CHUNK
    c_slabs = _CHUNK // _SUB
    mesh = plsc.VectorSubcoreMesh(core_axis_name="c", subcore_axis_name="s")

    @functools.partial(
        pl.kernel,
        mesh=mesh,
        compiler_params=pltpu.CompilerParams(use_tc_tiling_on_sc=True),
        out_type=[
            jax.ShapeDtypeStruct((B // _SUB, _SUB, D), jnp.float32),
            jax.ShapeDtypeStruct((B // _SUB, _SUB, D), jnp.float32),
        ],
        scratch_types=[
            pltpu.VMEM((b_per_w,), jnp.int32),
            pltpu.VMEM((b_per_w,), jnp.int32),
            pltpu.VMEM((c_slabs, _SUB, D), jnp.float32),
            pltpu.VMEM((c_slabs, _SUB, D), jnp.float32),
            pltpu.SemaphoreType.DMA,
            pltpu.SemaphoreType.DMA,
        ],
    )
    def k(uidx_hbm, iidx_hbm, ut_hbm, it_hbm, ue_hbm, ie_hbm,
          uix, iix, ubuf, ibuf, sem_u, sem_i):
        wid = lax.axis_index("s") * _NC + lax.axis_index("c")
        base = wid * b_per_w

        pltpu.sync_copy(uidx_hbm.at[pl.ds(base, b_per_w)], uix)
        pltpu.sync_copy(iidx_hbm.at[pl.ds(base, b_per_w)], iix)

        def issue_rows(tab_hbm, ixv, buf, sem, h):
            def body(g, carry):
                vec = ixv[pl.ds(h * _CHUNK + g * 16, 16)]
                for rr in range(16):
                    r = _extract(vec, rr)
                    q = lax.shift_right_logical(r, 3)
                    s = lax.bitwise_and(r, 7)
                    pltpu.async_copy(
                        tab_hbm.at[q, s],
                        buf.at[2 * g + rr // _SUB, rr % _SUB],
                        sem)
                return carry
            lax.fori_loop(0, _CHUNK // 16, body, 0)

        for h in range(n_chunks):
            issue_rows(ut_hbm, uix, ubuf, sem_u, h)
            issue_rows(it_hbm, iix, ibuf, sem_i, h)
            pltpu.make_async_copy(
                ut_hbm.at[pl.ds(0, c_slabs)], ubuf, sem_u).wait()
            pltpu.make_async_copy(
                it_hbm.at[pl.ds(0, c_slabs)], ibuf, sem_i).wait()
            out_off = wid * (b_per_w // _SUB) + h * c_slabs
            pltpu.sync_copy(ubuf, ue_hbm.at[pl.ds(out_off, c_slabs)])
            pltpu.sync_copy(ibuf, ie_hbm.at[pl.ds(out_off, c_slabs)])

    return k(uidx, iidx, ut3, it3)


def _dot_body(u_ref, i_ref, b_ref, o_ref):
    o_ref[...] = (
        jnp.sum(u_ref[...] * i_ref[...], axis=1, keepdims=True) + b_ref[0]
    )


@functools.partial(jax.jit, static_argnums=(3, 4, 5))
def _dot_tc(ue, ie, bias, B, D, blk):
    return pl.pallas_call(
        _dot_body,
        grid=(B // blk,),
        in_specs=[
            pl.BlockSpec((blk, D), lambda i: (i, 0)),
            pl.BlockSpec((blk, D), lambda i: (i, 0)),
            pl.BlockSpec(memory_space=pltpu.SMEM),
        ],
        out_specs=pl.BlockSpec((blk, 1), lambda i: (i, 0)),
        out_shape=jax.ShapeDtypeStruct((B, 1), jnp.float32),
    )(ue, ie, bias)


def kernel(x, user_table, item_table, bias):
    B = x.shape[0]
    V, D = user_table.shape
    uidx = x[:, 0]
    iidx = x[:, 1]
    ue3, ie3 = _gather_sc(
        uidx, iidx,
        user_table.reshape(V // _SUB, _SUB, D),
        item_table.reshape(V // _SUB, _SUB, D),
        V, B, D)
    ue = ue3.reshape(B, D)
    ie = ie3.reshape(B, D)
    out = _dot_tc(ue, ie, bias, B, D, 2048)
    return out, ue, ie
